# baseline (speedup 1.0000x reference)
import jax
import jax.numpy as jnp
from jax import lax
from jax.experimental import pallas as pl
from jax.experimental.pallas import tpu as pltpu
from jax.experimental.pallas import tpu_sc as plsc

_EMBED = 64
_TVOC = 1000000
_DVOC = 1000
_B, _L = 4096, 200
_N = _B * _L
_NC, _NS = 2, 16
_NW = _NC * _NS
_PER_W = _N // _NW
_G = 128
_STEPS = _PER_W // _G
_E2 = 2 * _EMBED
_NBUF = 4
_V = 16

_PROWS = 31248
_PR = 168
_PSTEPS = _PROWS // _PR
_PBUF = 3
_PTAIL = _TVOC - _NW * _PROWS


def _pad_body(tab_hbm, slot_hbm, nbuf_v, buf_v, sem_r, sem_w):
    c = lax.axis_index("c")
    s = lax.axis_index("s")
    wid = s * _NC + c
    base = wid * _PROWS

    zeros = jnp.zeros((_V,), jnp.float32)

    @pl.loop(0, _PR)
    def _(r):
        for b in range(_PBUF):
            for q in range(_EMBED // _V):
                buf_v[b, r, pl.ds(_EMBED + q * _V, _V)] = zeros

    def rd(j, b):
        return pltpu.make_async_copy(
            tab_hbm.at[pl.ds(base + j * _PR, _PR)],
            nbuf_v.at[b],
            sem_r.at[b])

    def wr(j, b):
        return pltpu.make_async_copy(
            buf_v.at[b], slot_hbm.at[pl.ds(base + j * _PR, _PR)],
            sem_w.at[b])

    def interleave(b):
        @pl.loop(0, _PR)
        def _(r):
            for q in range(_EMBED // _V):
                buf_v[b, r, pl.ds(q * _V, _V)] = nbuf_v[b, r, pl.ds(q * _V, _V)]

    rd(0, 0).start()
    rd(1, 1).start()
    rd(0, 0).wait()
    interleave(0)
    wr(0, 0).start()
    rd(2, 2).start()
    rd(1, 1).wait()
    interleave(1)
    wr(1, 1).start()

    @pl.loop(_PBUF, _PSTEPS, step=_PBUF)
    def _(j0):
        for b in range(_PBUF):
            j = j0 + b
            wr(j - _PBUF, b).wait()
            rd(j, b).start()
            b1 = (b - 1) % _PBUF
            rd(j - 1, b1).wait()
            interleave(b1)
            wr(j - 1, b1).start()

    rd(_PSTEPS - 1, _PBUF - 1).wait()
    interleave(_PBUF - 1)
    wr(_PSTEPS - 1, _PBUF - 1).start()
    wr(_PSTEPS - 3, 0).wait()
    wr(_PSTEPS - 2, 1).wait()
    wr(_PSTEPS - 1, 2).wait()

    @pl.when(wid == 0)
    def _():
        tail = _NW * _PROWS
        cp = pltpu.make_async_copy(
            tab_hbm.at[pl.ds(tail, _PTAIL)],
            nbuf_v.at[0, pl.ds(0, _PTAIL)],
            sem_r.at[0])
        cp.start()
        cp.wait()

        @pl.loop(0, _PTAIL)
        def _(r):
            for q in range(_EMBED // _V):
                buf_v[0, r, pl.ds(q * _V, _V)] = nbuf_v[0, r, pl.ds(q * _V, _V)]

        cpw = pltpu.make_async_copy(
            buf_v.at[0, pl.ds(0, _PTAIL)],
            slot_hbm.at[pl.ds(tail, _PTAIL)],
            sem_w.at[0])
        cpw.start()
        cpw.wait()


def _body(tok_idx_hbm, diac_idx_hbm, tok_tab_hbm, diac_tab_hbm, out_hbm,
          idx_t_v, idx_d_v, rows_v, dia_sh, sem_t, sem_a, sem_w):
    c = lax.axis_index("c")
    s = lax.axis_index("s")
    wid = s * _NC + c
    base = wid * _PER_W

    @pl.when(s == 0)
    def _():
        pltpu.sync_copy(diac_tab_hbm, dia_sh)

    plsc.subcore_barrier()

    pltpu.sync_copy(tok_idx_hbm.at[pl.ds(base, _PER_W)], idx_t_v)
    pltpu.sync_copy(diac_idx_hbm.at[pl.ds(base, _PER_W)], idx_d_v)

    def tok_copy(j, b):
        return pltpu.make_async_copy(
            tok_tab_hbm.at[idx_t_v.at[pl.ds(j * _G, _G)]], rows_v.at[b],
            sem_t.at[b])

    def add_copy(j, b):
        return pltpu.make_async_copy(
            dia_sh.at[idx_d_v.at[pl.ds(j * _G, _G)]], rows_v.at[b],
            sem_a.at[b])

    def wr_copy(j, b):
        return pltpu.make_async_copy(
            rows_v.at[b], out_hbm.at[pl.ds(base + j * _G, _G)], sem_w.at[b])

    tok_copy(0, 0).start()
    tok_copy(1, 1).start()
    tok_copy(0, 0).wait()
    add_copy(0, 0).start(add=True)
    tok_copy(2, 2).start()
    tok_copy(1, 1).wait()
    add_copy(1, 1).start(add=True)
    add_copy(0, 0).wait()
    wr_copy(0, 0).start()
    tok_copy(3, 3).start()
    tok_copy(2, 2).wait()
    add_copy(2, 2).start(add=True)
    add_copy(1, 1).wait()
    wr_copy(1, 1).start()

    @pl.loop(4, _STEPS, step=_NBUF)
    def _(j0):
        for b in range(_NBUF):
            j = j0 + b
            wr_copy(j - _NBUF, b).wait()
            tok_copy(j, b).start()
            tok_copy(j - 1, (b - 1) % _NBUF).wait()
            add_copy(j - 1, (b - 1) % _NBUF).start(add=True)
            add_copy(j - 2, (b - 2) % _NBUF).wait()
            wr_copy(j - 2, (b - 2) % _NBUF).start()

    tok_copy(_STEPS - 1, 3).wait()
    add_copy(_STEPS - 1, 3).start(add=True)
    add_copy(_STEPS - 2, 2).wait()
    wr_copy(_STEPS - 2, 2).start()
    add_copy(_STEPS - 1, 3).wait()
    wr_copy(_STEPS - 1, 3).start()
    wr_copy(_STEPS - 4, 0).wait()
    wr_copy(_STEPS - 3, 1).wait()
    wr_copy(_STEPS - 2, 2).wait()
    wr_copy(_STEPS - 1, 3).wait()


def kernel(token_inputs, diac_inputs, token_table, diac_table):
    tok_idx = token_inputs.reshape(-1)
    diac_idx = diac_inputs.reshape(-1)
    mesh = plsc.VectorSubcoreMesh(core_axis_name="c", subcore_axis_name="s")

    pad_k = pl.kernel(
        _pad_body,
        mesh=mesh,
        out_type=jax.ShapeDtypeStruct((_TVOC, _E2), jnp.float32),
        scratch_types=[
            pltpu.VMEM((_PBUF, _PR, _EMBED), jnp.float32),
            pltpu.VMEM((_PBUF, _PR, _E2), jnp.float32),
            pltpu.SemaphoreType.DMA((_PBUF,)),
            pltpu.SemaphoreType.DMA((_PBUF,)),
        ],
    )
    tok_tab = pad_k(token_table)

    diac_tab = jnp.pad(diac_table, ((0, 0), (_EMBED, 0)))

    k = pl.kernel(
        _body,
        mesh=mesh,
        out_type=jax.ShapeDtypeStruct((_N, _E2), jnp.float32),
        scratch_types=[
            pltpu.VMEM((_PER_W,), jnp.int32),
            pltpu.VMEM((_PER_W,), jnp.int32),
            pltpu.VMEM((_NBUF, _G, _E2), jnp.float32),
            pltpu.VMEM_SHARED((_DVOC, _E2), jnp.float32),
            pltpu.SemaphoreType.DMA((_NBUF,)),
            pltpu.SemaphoreType.DMA((_NBUF,)),
            pltpu.SemaphoreType.DMA((_NBUF,)),
        ],
    )
    out = k(tok_idx, diac_idx, tok_tab, diac_tab)
    return out.reshape(_B, _L, _E2)


# speedup vs baseline: 1.1897x; 1.1897x over previous
import jax
import jax.numpy as jnp
from jax import lax
from jax.experimental import pallas as pl
from jax.experimental.pallas import tpu as pltpu
from jax.experimental.pallas import tpu_sc as plsc

_EMBED = 64
_DVOC = 1000
_B, _L = 4096, 200
_N = _B * _L
_NC, _NS = 2, 16
_NW = _NC * _NS
_PER_W = _N // _NW
_G = 128
_STEPS = _PER_W // _G
_E2 = 2 * _EMBED
_NBUF = 4


def _body(tok_idx_hbm, diac_idx_hbm, tok_tab_hbm, diac_tab_hbm, out_hbm,
          idx_t_v, idx_d_v, rows_v, dia_sh, sem_t, sem_a, sem_w):
    c = lax.axis_index("c")
    s = lax.axis_index("s")
    wid = s * _NC + c
    base = wid * _PER_W

    @pl.when(s == 0)
    def _():
        pltpu.sync_copy(diac_tab_hbm, dia_sh)

    plsc.subcore_barrier()

    pltpu.sync_copy(tok_idx_hbm.at[pl.ds(base, _PER_W)], idx_t_v)
    pltpu.sync_copy(diac_idx_hbm.at[pl.ds(base, _PER_W)], idx_d_v)

    def tok_copy(j, b):
        return pltpu.make_async_copy(
            tok_tab_hbm.at[idx_t_v.at[pl.ds(j * _G, _G)]], rows_v.at[b],
            sem_t.at[b])

    def add_copy(j, b):
        return pltpu.make_async_copy(
            dia_sh.at[idx_d_v.at[pl.ds(j * _G, _G)]], rows_v.at[b],
            sem_a.at[b])

    def wr_copy(j, b):
        return pltpu.make_async_copy(
            rows_v.at[b], out_hbm.at[pl.ds(base + j * _G, _G)], sem_w.at[b])

    tok_copy(0, 0).start()
    tok_copy(1, 1).start()
    tok_copy(0, 0).wait()
    add_copy(0, 0).start(add=True)
    tok_copy(2, 2).start()
    tok_copy(1, 1).wait()
    add_copy(1, 1).start(add=True)
    add_copy(0, 0).wait()
    wr_copy(0, 0).start()
    tok_copy(3, 3).start()
    tok_copy(2, 2).wait()
    add_copy(2, 2).start(add=True)
    add_copy(1, 1).wait()
    wr_copy(1, 1).start()

    @pl.loop(4, _STEPS, step=_NBUF)
    def _(j0):
        for b in range(_NBUF):
            j = j0 + b
            wr_copy(j - _NBUF, b).wait()
            tok_copy(j, b).start()
            tok_copy(j - 1, (b - 1) % _NBUF).wait()
            add_copy(j - 1, (b - 1) % _NBUF).start(add=True)
            add_copy(j - 2, (b - 2) % _NBUF).wait()
            wr_copy(j - 2, (b - 2) % _NBUF).start()

    tok_copy(_STEPS - 1, 3).wait()
    add_copy(_STEPS - 1, 3).start(add=True)
    add_copy(_STEPS - 2, 2).wait()
    wr_copy(_STEPS - 2, 2).start()
    add_copy(_STEPS - 1, 3).wait()
    wr_copy(_STEPS - 1, 3).start()
    wr_copy(_STEPS - 4, 0).wait()
    wr_copy(_STEPS - 3, 1).wait()
    wr_copy(_STEPS - 2, 2).wait()
    wr_copy(_STEPS - 1, 3).wait()


def kernel(token_inputs, diac_inputs, token_table, diac_table):
    tok_idx = token_inputs.reshape(-1)
    diac_idx = diac_inputs.reshape(-1)
    tok_tab = jnp.concatenate(
        [token_table, jnp.zeros_like(token_table)], axis=1)
    diac_tab = jnp.pad(diac_table, ((0, 0), (_EMBED, 0)))
    mesh = plsc.VectorSubcoreMesh(core_axis_name="c", subcore_axis_name="s")
    k = pl.kernel(
        _body,
        mesh=mesh,
        out_type=jax.ShapeDtypeStruct((_N, _E2), jnp.float32),
        scratch_types=[
            pltpu.VMEM((_PER_W,), jnp.int32),
            pltpu.VMEM((_PER_W,), jnp.int32),
            pltpu.VMEM((_NBUF, _G, _E2), jnp.float32),
            pltpu.VMEM_SHARED((_DVOC, _E2), jnp.float32),
            pltpu.SemaphoreType.DMA((_NBUF,)),
            pltpu.SemaphoreType.DMA((_NBUF,)),
            pltpu.SemaphoreType.DMA((_NBUF,)),
        ],
    )
    out = k(tok_idx, diac_idx, tok_tab, diac_tab)
    return out.reshape(_B, _L, _E2)
